# baseline (speedup 1.0000x reference)
import functools

import jax
import jax.numpy as jnp
from jax import lax
from jax.experimental import pallas as pl
from jax.experimental.pallas import tpu as pltpu
from jax.experimental.pallas import tpu_sc as plsc

_NC = 2
_NS = 16
_NW = _NC * _NS
_N = 10000
_H = 128
_NPAD = 10240
_RPT = _NPAD // _NS
_CH = 128


def _make_agg(nchunk):
    mesh = plsc.VectorSubcoreMesh(core_axis_name="c", subcore_axis_name="s")
    out_type = jax.ShapeDtypeStruct((_NC, _NPAD, _H), jnp.float32)
    scratch = [
        pltpu.VMEM((nchunk, _CH), jnp.int32),
        pltpu.VMEM((_CH,), jnp.int32),
        pltpu.VMEM((_CH, _H), jnp.float32),
        pltpu.VMEM((16, _H), jnp.float32),
        pltpu.VMEM_SHARED((_NPAD, _H), jnp.float32),
        pltpu.SemaphoreType.DMA,
    ]

    def body(x_hbm, src_hbm, dst_hbm, psum, sidx, didx0, rows, zbuf,
             acc, semA):
        c = lax.axis_index("c")
        s = lax.axis_index("s")
        wid = s * _NC + c

        zv = jnp.zeros((16,), jnp.float32)
        for r in range(16):
            for g in range(8):
                zbuf[r, pl.ds(g * 16, 16)] = zv
        base = s * _RPT
        for j in range(_RPT // 16):
            pltpu.sync_copy(zbuf, acc.at[pl.ds(base + j * 16, 16)])
        pltpu.sync_copy(src_hbm.at[wid], sidx)
        plsc.subcore_barrier()

        def chunk(k, carry):
            pltpu.sync_copy(dst_hbm.at[wid, k], didx0)
            pltpu.async_copy(x_hbm.at[sidx.at[k]], rows, semA).wait()
            pltpu.sync_copy(rows, acc.at[didx0], add=True)
            return carry

        lax.fori_loop(0, nchunk, chunk, 0)
        plsc.subcore_barrier()

        pltpu.sync_copy(acc.at[pl.ds(base, _RPT)],
                        psum.at[c, pl.ds(base, _RPT)])

    return pl.kernel(body, out_type=out_type, mesh=mesh,
                     scratch_types=scratch)


def _make_cnt(nchunk):
    mesh = plsc.VectorSubcoreMesh(core_axis_name="c", subcore_axis_name="s")
    out_type = jax.ShapeDtypeStruct((_NC, _NPAD, _H), jnp.float32)
    scratch = [
        pltpu.VMEM((_CH,), jnp.int32),
        pltpu.VMEM((_CH, _H), jnp.float32),
        pltpu.VMEM((16, _H), jnp.float32),
        pltpu.VMEM_SHARED((_NPAD, _H), jnp.float32),
    ]

    def body(dst_hbm, pcnt, didx0, ones, zbuf, cac):
        c = lax.axis_index("c")
        s = lax.axis_index("s")
        wid = s * _NC + c

        zv = jnp.zeros((16,), jnp.float32)
        for r in range(16):
            for g in range(8):
                zbuf[r, pl.ds(g * 16, 16)] = zv
        ov = jnp.ones((16,), jnp.float32)

        def orow(r, carry):
            for g in range(8):
                ones[r, pl.ds(g * 16, 16)] = ov
            return carry

        lax.fori_loop(0, _CH, orow, 0)

        base = s * _RPT
        for j in range(_RPT // 16):
            pltpu.sync_copy(zbuf, cac.at[pl.ds(base + j * 16, 16)])
        plsc.subcore_barrier()

        def chunk(k, carry):
            pltpu.sync_copy(dst_hbm.at[wid, k], didx0)
            pltpu.sync_copy(ones, cac.at[didx0], add=True)
            return carry

        lax.fori_loop(0, nchunk, chunk, 0)
        plsc.subcore_barrier()

        pltpu.sync_copy(cac.at[pl.ds(base, _RPT)],
                        pcnt.at[c, pl.ds(base, _RPT)])

    return pl.kernel(body, out_type=out_type, mesh=mesh,
                     scratch_types=scratch)


def _make_gather(nchunk):
    mesh = plsc.VectorSubcoreMesh(core_axis_name="c", subcore_axis_name="s")
    out_type = jax.ShapeDtypeStruct((_NW * nchunk * _CH, _H), jnp.float32)
    scratch = [
        pltpu.VMEM((nchunk, _CH), jnp.int32),
        pltpu.VMEM((_CH, _H), jnp.float32),
        pltpu.VMEM((_CH, _H), jnp.float32),
        pltpu.SemaphoreType.DMA,
        pltpu.SemaphoreType.DMA,
    ]

    def body(x_hbm, idx_hbm, out_hbm, idxv, rows0, rows1, sem0, sem1):
        c = lax.axis_index("c")
        s = lax.axis_index("s")
        wid = s * _NC + c
        pltpu.sync_copy(idx_hbm.at[wid], idxv)
        for p in range(nchunk // 2):
            k0 = 2 * p
            k1 = k0 + 1
            cp0 = pltpu.async_copy(x_hbm.at[idxv.at[k0]], rows0, sem0)
            cp1 = pltpu.async_copy(x_hbm.at[idxv.at[k1]], rows1, sem1)
            cp0.wait()
            pltpu.sync_copy(
                rows0, out_hbm.at[pl.ds((wid * nchunk + k0) * _CH, _CH)])
            cp1.wait()
            pltpu.sync_copy(
                rows1, out_hbm.at[pl.ds((wid * nchunk + k1) * _CH, _CH)])

    return pl.kernel(body, out_type=out_type, mesh=mesh, scratch_types=scratch)


def _enc_body(x_ref, w1_ref, b1_ref, w2_ref, b2_ref, o_ref):
    h = jnp.dot(x_ref[...], w1_ref[...], preferred_element_type=jnp.float32)
    h = jnp.maximum(h + b1_ref[...], 0.0)
    o_ref[...] = jnp.dot(h, w2_ref[...],
                         preferred_element_type=jnp.float32) + b2_ref[...]


def _conv_body(x_ref, ps_ref, pc_ref, wl_ref, bl_ref, wr_ref, o_ref):
    x = x_ref[...]
    agg = ps_ref[0, :_N, :] + ps_ref[1, :_N, :]
    cnt = pc_ref[0, :_N, 0:1] + pc_ref[1, :_N, 0:1]
    agg = agg / jnp.maximum(cnt, 1.0)
    t = jnp.dot(agg, wl_ref[...], preferred_element_type=jnp.float32)
    t = t + bl_ref[...]
    t = t - jnp.mean(t, axis=0, keepdims=True)
    t = t + jnp.dot(x, wr_ref[...], preferred_element_type=jnp.float32)
    xn = x + jnp.maximum(t, 0.0)
    o_ref[...] = xn - jnp.mean(xn, axis=0, keepdims=True)


_KNN_Q = 400


def _knn_body(py_ref, pxt_ref, om0, om1, om2, oi0, oi1, oi2):
    d2 = None
    for d in range(3):
        diff = py_ref[:, d:d + 1] - pxt_ref[d:d + 1, :]
        d2 = diff * diff if d2 is None else d2 + diff * diff
    iota = lax.broadcasted_iota(jnp.int32, (_KNN_Q, _N), 1)
    outs_m = (om0, om1, om2)
    outs_i = (oi0, oi1, oi2)
    for r in range(3):
        m = jnp.min(d2, axis=1, keepdims=True)
        idx = jnp.min(jnp.where(d2 <= m, iota, 10000000), axis=1,
                      keepdims=True)
        outs_m[r][...] = m
        outs_i[r][...] = idx
        if r < 2:
            d2 = jnp.where(iota == idx, jnp.float32(jnp.inf), d2)


def _interp_body(g_ref, m0_ref, m1_ref, m2_ref, o_ref):
    w0 = 1.0 / jnp.clip(m0_ref[...], 1e-16, None)
    w1 = 1.0 / jnp.clip(m1_ref[...], 1e-16, None)
    w2 = 1.0 / jnp.clip(m2_ref[...], 1e-16, None)
    num = w0 * g_ref[0] + w1 * g_ref[1] + w2 * g_ref[2]
    o_ref[...] = num / (w0 + w1 + w2)


def _pad_edges(e):
    nch = -(-e.shape[1] // (_NW * _CH))
    nch = nch + (nch % 2)
    epw = nch * _CH
    epad = _NW * epw
    pad = epad - e.shape[1]
    src = jnp.concatenate([e[0].astype(jnp.int32),
                           jnp.zeros((pad,), jnp.int32)])
    dst = jnp.concatenate([e[1].astype(jnp.int32),
                           jnp.full((pad,), _N, jnp.int32)])
    return (src.reshape(_NW, epw // _CH, _CH),
            dst.reshape(_NW, epw // _CH, _CH), epw // _CH)


def kernel(l_pos1, l_y1, l_e1, h_pos1, h_e1, Wenc1, benc1, Wenc2, benc2,
           Wl1, bl1, Wr1, Wl2, bl2, Wr2):
    f32 = jnp.float32
    x19 = jnp.concatenate([l_y1, l_pos1], axis=-1)

    enc = pl.pallas_call(
        _enc_body,
        out_shape=jax.ShapeDtypeStruct((_N, _H), f32),
    )(x19, Wenc1.T, benc1[None, :], Wenc2.T, benc2[None, :])

    conv = pl.pallas_call(
        _conv_body,
        out_shape=jax.ShapeDtypeStruct((_N, _H), f32),
    )

    src_l, dst_l, nchunk = _pad_edges(l_e1)
    src_h, dst_h, _ = _pad_edges(h_e1)
    cntk = _make_cnt(nchunk)
    agg = _make_agg(nchunk)

    x = enc
    for gph, (src, dst, Wl, bl, Wr) in enumerate(
            ((src_l, dst_l, Wl1, bl1, Wr1), (src_h, dst_h, Wl2, bl2, Wr2))):
        pc = cntk(dst)
        ps = agg(x, src, dst)
        x = conv(x, ps, pc, Wl[0].T, bl[0][None, :], Wr[0].T)
        ps2 = agg(x, src, dst)
        x = conv(x, ps2, pc, Wl[1].T, bl[1][None, :], Wr[1].T)

        if gph == 0:
            nb = _N // _KNN_Q
            knn_outs = pl.pallas_call(
                _knn_body,
                grid=(nb,),
                in_specs=[
                    pl.BlockSpec((_KNN_Q, 3), lambda i: (i, 0)),
                    pl.BlockSpec((3, _N), lambda i: (0, 0)),
                ],
                out_specs=[pl.BlockSpec((_KNN_Q, 1), lambda i: (i, 0))] * 6,
                out_shape=[jax.ShapeDtypeStruct((_N, 1), f32)] * 3
                + [jax.ShapeDtypeStruct((_N, 1), jnp.int32)] * 3,
            )(h_pos1, l_pos1.T)
            m0, m1, m2, i0, i1, i2 = knn_outs
            idx_flat = jnp.concatenate([i0[:, 0], i1[:, 0], i2[:, 0]])
            gchunk = -(-idx_flat.shape[0] // (_NW * _CH))
            gpad = _NW * gchunk * _CH - idx_flat.shape[0]
            idx_flat = jnp.concatenate([idx_flat,
                                        jnp.zeros((gpad,), jnp.int32)])
            g = _make_gather(gchunk)(
                x, idx_flat.reshape(_NW, gchunk, _CH))
            g3 = g[:3 * _N].reshape(3, _N, _H)
            x = pl.pallas_call(
                _interp_body,
                grid=(10,),
                in_specs=[
                    pl.BlockSpec((3, _N // 10, _H), lambda i: (0, i, 0)),
                    pl.BlockSpec((_N // 10, 1), lambda i: (i, 0)),
                    pl.BlockSpec((_N // 10, 1), lambda i: (i, 0)),
                    pl.BlockSpec((_N // 10, 1), lambda i: (i, 0)),
                ],
                out_specs=pl.BlockSpec((_N // 10, _H), lambda i: (i, 0)),
                out_shape=jax.ShapeDtypeStruct((_N, _H), f32),
            )(g3, m0, m1, m2)

    return x


# speedup vs baseline: 1.4245x; 1.4245x over previous
import functools

import jax
import jax.numpy as jnp
from jax import lax
from jax.experimental import pallas as pl
from jax.experimental.pallas import tpu as pltpu
from jax.experimental.pallas import tpu_sc as plsc

_NC = 2
_NS = 16
_NW = _NC * _NS
_N = 10000
_H = 128
_NPAD = 10240
_RPT = _NPAD // _NS
_CH = 128


def _make_agg(nchunk):
    mesh = plsc.VectorSubcoreMesh(core_axis_name="c", subcore_axis_name="s")
    out_type = jax.ShapeDtypeStruct((_NC, _NPAD, _H), jnp.float32)
    scratch = [
        pltpu.VMEM((nchunk, _CH), jnp.int32),
        pltpu.VMEM((_CH,), jnp.int32),
        pltpu.VMEM((_CH, _H), jnp.float32),
        pltpu.VMEM((16, _H), jnp.float32),
        pltpu.VMEM_SHARED((_NPAD, _H), jnp.float32),
        pltpu.SemaphoreType.DMA,
    ]

    def body(x_hbm, src_hbm, dst_hbm, psum, sidx, didx0, rows, zbuf,
             acc, semA):
        c = lax.axis_index("c")
        s = lax.axis_index("s")
        wid = s * _NC + c

        zv = jnp.zeros((16,), jnp.float32)
        for r in range(16):
            for g in range(8):
                zbuf[r, pl.ds(g * 16, 16)] = zv
        base = s * _RPT
        for j in range(_RPT // 16):
            pltpu.sync_copy(zbuf, acc.at[pl.ds(base + j * 16, 16)])
        pltpu.sync_copy(src_hbm.at[wid], sidx)
        plsc.subcore_barrier()

        def chunk(k, carry):
            pltpu.sync_copy(dst_hbm.at[wid, k], didx0)
            pltpu.async_copy(x_hbm.at[sidx.at[k]], rows, semA).wait()
            pltpu.sync_copy(rows, acc.at[didx0], add=True)
            return carry

        lax.fori_loop(0, nchunk, chunk, 0)
        plsc.subcore_barrier()

        pltpu.sync_copy(acc.at[pl.ds(base, _RPT)],
                        psum.at[c, pl.ds(base, _RPT)])

    return pl.kernel(body, out_type=out_type, mesh=mesh,
                     scratch_types=scratch)


def _make_cnt(nchunk):
    mesh = plsc.VectorSubcoreMesh(core_axis_name="c", subcore_axis_name="s")
    out_type = jax.ShapeDtypeStruct((_NC, _NPAD, _H), jnp.float32)
    scratch = [
        pltpu.VMEM((_CH,), jnp.int32),
        pltpu.VMEM((_CH, _H), jnp.float32),
        pltpu.VMEM((16, _H), jnp.float32),
        pltpu.VMEM_SHARED((_NPAD, _H), jnp.float32),
    ]

    def body(dst_hbm, pcnt, didx0, ones, zbuf, cac):
        c = lax.axis_index("c")
        s = lax.axis_index("s")
        wid = s * _NC + c

        zv = jnp.zeros((16,), jnp.float32)
        for r in range(16):
            for g in range(8):
                zbuf[r, pl.ds(g * 16, 16)] = zv
        ov = jnp.ones((16,), jnp.float32)

        def orow(r, carry):
            for g in range(8):
                ones[r, pl.ds(g * 16, 16)] = ov
            return carry

        lax.fori_loop(0, _CH, orow, 0)

        base = s * _RPT
        for j in range(_RPT // 16):
            pltpu.sync_copy(zbuf, cac.at[pl.ds(base + j * 16, 16)])
        plsc.subcore_barrier()

        def chunk(k, carry):
            pltpu.sync_copy(dst_hbm.at[wid, k], didx0)
            pltpu.sync_copy(ones, cac.at[didx0], add=True)
            return carry

        lax.fori_loop(0, nchunk, chunk, 0)
        plsc.subcore_barrier()

        pltpu.sync_copy(cac.at[pl.ds(base, _RPT)],
                        pcnt.at[c, pl.ds(base, _RPT)])

    return pl.kernel(body, out_type=out_type, mesh=mesh,
                     scratch_types=scratch)


def _make_gather(nchunk):
    mesh = plsc.VectorSubcoreMesh(core_axis_name="c", subcore_axis_name="s")
    out_type = jax.ShapeDtypeStruct((_NW * nchunk * _CH, _H), jnp.float32)
    scratch = [
        pltpu.VMEM((nchunk, _CH), jnp.int32),
        pltpu.VMEM((_CH, _H), jnp.float32),
        pltpu.VMEM((_CH, _H), jnp.float32),
        pltpu.SemaphoreType.DMA,
        pltpu.SemaphoreType.DMA,
    ]

    def body(x_hbm, idx_hbm, out_hbm, idxv, rows0, rows1, sem0, sem1):
        c = lax.axis_index("c")
        s = lax.axis_index("s")
        wid = s * _NC + c
        pltpu.sync_copy(idx_hbm.at[wid], idxv)
        for p in range(nchunk // 2):
            k0 = 2 * p
            k1 = k0 + 1
            cp0 = pltpu.async_copy(x_hbm.at[idxv.at[k0]], rows0, sem0)
            cp1 = pltpu.async_copy(x_hbm.at[idxv.at[k1]], rows1, sem1)
            cp0.wait()
            pltpu.sync_copy(
                rows0, out_hbm.at[pl.ds((wid * nchunk + k0) * _CH, _CH)])
            cp1.wait()
            pltpu.sync_copy(
                rows1, out_hbm.at[pl.ds((wid * nchunk + k1) * _CH, _CH)])

    return pl.kernel(body, out_type=out_type, mesh=mesh, scratch_types=scratch)


def _enc_body(x_ref, w1_ref, b1_ref, w2_ref, b2_ref, o_ref):
    h = jnp.dot(x_ref[...], w1_ref[...], preferred_element_type=jnp.float32)
    h = jnp.maximum(h + b1_ref[...], 0.0)
    o_ref[...] = jnp.dot(h, w2_ref[...],
                         preferred_element_type=jnp.float32) + b2_ref[...]


def _conv_body(x_ref, ps_ref, pc_ref, wl_ref, bl_ref, wr_ref, o_ref):
    x = x_ref[...]
    agg = ps_ref[0, :_N, :] + ps_ref[1, :_N, :]
    cnt = pc_ref[0, :_N, 0:1] + pc_ref[1, :_N, 0:1]
    agg = agg / jnp.maximum(cnt, 1.0)
    t = jnp.dot(agg, wl_ref[...], preferred_element_type=jnp.float32)
    t = t + bl_ref[...]
    t = t - jnp.mean(t, axis=0, keepdims=True)
    t = t + jnp.dot(x, wr_ref[...], preferred_element_type=jnp.float32)
    xn = x + jnp.maximum(t, 0.0)
    o_ref[...] = xn - jnp.mean(xn, axis=0, keepdims=True)


_KNN_Q = 400


def _knn_body(py_ref, pxt_ref, om0, om1, om2, oi0, oi1, oi2):
    d2 = None
    for d in range(3):
        diff = py_ref[:, d:d + 1] - pxt_ref[d:d + 1, :]
        d2 = diff * diff if d2 is None else d2 + diff * diff
    iota = lax.broadcasted_iota(jnp.int32, (_KNN_Q, _N), 1)
    outs_m = (om0, om1, om2)
    outs_i = (oi0, oi1, oi2)
    for r in range(3):
        m = jnp.min(d2, axis=1, keepdims=True)
        idx = jnp.min(jnp.where(d2 <= m, iota, 10000000), axis=1,
                      keepdims=True)
        outs_m[r][...] = m
        outs_i[r][...] = idx
        if r < 2:
            d2 = jnp.where(iota == idx, jnp.float32(jnp.inf), d2)


def _interp_body(g_ref, m0_ref, m1_ref, m2_ref, o_ref):
    w0 = 1.0 / jnp.clip(m0_ref[...], 1e-16, None)
    w1 = 1.0 / jnp.clip(m1_ref[...], 1e-16, None)
    w2 = 1.0 / jnp.clip(m2_ref[...], 1e-16, None)
    num = w0 * g_ref[0] + w1 * g_ref[1] + w2 * g_ref[2]
    o_ref[...] = num / (w0 + w1 + w2)


def _pad_edges(e):
    epw = -(-e.shape[1] // (_NW * _CH)) * _CH
    epad = _NW * epw
    pad = epad - e.shape[1]
    src = jnp.concatenate([e[0].astype(jnp.int32),
                           jnp.zeros((pad,), jnp.int32)])
    dst = jnp.concatenate([e[1].astype(jnp.int32),
                           jnp.full((pad,), _N, jnp.int32)])
    return (src.reshape(_NW, epw // _CH, _CH),
            dst.reshape(_NW, epw // _CH, _CH), epw // _CH)


def kernel(l_pos1, l_y1, l_e1, h_pos1, h_e1, Wenc1, benc1, Wenc2, benc2,
           Wl1, bl1, Wr1, Wl2, bl2, Wr2):
    f32 = jnp.float32
    x19 = jnp.concatenate([l_y1, l_pos1], axis=-1)

    enc = pl.pallas_call(
        _enc_body,
        out_shape=jax.ShapeDtypeStruct((_N, _H), f32),
    )(x19, Wenc1.T, benc1[None, :], Wenc2.T, benc2[None, :])

    conv = pl.pallas_call(
        _conv_body,
        out_shape=jax.ShapeDtypeStruct((_N, _H), f32),
    )

    src_l, dst_l, nchunk = _pad_edges(l_e1)
    src_h, dst_h, _ = _pad_edges(h_e1)
    cntk = _make_cnt(nchunk)
    agg = _make_agg(nchunk)

    x = enc
    for gph, (src, dst, Wl, bl, Wr) in enumerate(
            ((src_l, dst_l, Wl1, bl1, Wr1), (src_h, dst_h, Wl2, bl2, Wr2))):
        pc = cntk(dst)
        ps = agg(x, src, dst)
        x = conv(x, ps, pc, Wl[0].T, bl[0][None, :], Wr[0].T)
        ps2 = agg(x, src, dst)
        x = conv(x, ps2, pc, Wl[1].T, bl[1][None, :], Wr[1].T)

        if gph == 0:
            nb = _N // _KNN_Q
            knn_outs = pl.pallas_call(
                _knn_body,
                grid=(nb,),
                in_specs=[
                    pl.BlockSpec((_KNN_Q, 3), lambda i: (i, 0)),
                    pl.BlockSpec((3, _N), lambda i: (0, 0)),
                ],
                out_specs=[pl.BlockSpec((_KNN_Q, 1), lambda i: (i, 0))] * 6,
                out_shape=[jax.ShapeDtypeStruct((_N, 1), f32)] * 3
                + [jax.ShapeDtypeStruct((_N, 1), jnp.int32)] * 3,
            )(h_pos1, l_pos1.T)
            m0, m1, m2, i0, i1, i2 = knn_outs
            idx_flat = jnp.concatenate([i0[:, 0], i1[:, 0], i2[:, 0]])
            gchunk = -(-idx_flat.shape[0] // (_NW * _CH))
            gpad = _NW * gchunk * _CH - idx_flat.shape[0]
            idx_flat = jnp.concatenate([idx_flat,
                                        jnp.zeros((gpad,), jnp.int32)])
            g = _make_gather(gchunk)(
                x, idx_flat.reshape(_NW, gchunk, _CH))
            g3 = g[:3 * _N].reshape(3, _N, _H)
            x = pl.pallas_call(
                _interp_body,
                grid=(10,),
                in_specs=[
                    pl.BlockSpec((3, _N // 10, _H), lambda i: (0, i, 0)),
                    pl.BlockSpec((_N // 10, 1), lambda i: (i, 0)),
                    pl.BlockSpec((_N // 10, 1), lambda i: (i, 0)),
                    pl.BlockSpec((_N // 10, 1), lambda i: (i, 0)),
                ],
                out_specs=pl.BlockSpec((_N // 10, _H), lambda i: (i, 0)),
                out_shape=jax.ShapeDtypeStruct((_N, _H), f32),
            )(g3, m0, m1, m2)

    return x
